# baseline (speedup 1.0000x reference)
import functools

import jax
import jax.numpy as jnp
from jax import lax
from jax.experimental import pallas as pl
from jax.experimental.pallas import tpu as pltpu
from jax.experimental.pallas import tpu_sc as plsc

N = 10000
F = 128
H = 64
E = 160000
C = 128

NC = 2
NS = 16
NW = NC * NS
CHK = 128
NCH = 40
NCHA = 80
E_PAD = NW * CHK * NCH
NP = 10240
ZR = NP // NS

_mesh = lambda: plsc.VectorSubcoreMesh(core_axis_name="c", subcore_axis_name="s",
                                       num_cores=NC, num_subcores=NS)


@functools.lru_cache(maxsize=None)
def _make_deg():
    @functools.partial(
        pl.kernel,
        out_type=jax.ShapeDtypeStruct((NC * NP,), jnp.float32),
        mesh=_mesh(),
        scratch_types=[
            pltpu.VMEM_SHARED((NP,), jnp.float32),
            pltpu.VMEM((CHK,), jnp.int32),
            pltpu.VMEM((CHK,), jnp.float32),
            pltpu.VMEM((ZR,), jnp.float32),
        ],
    )
    def deg_kernel(dst_hbm, ones_hbm, zeros_hbm, out_hbm, acc, didx, ones_v, vbuf):
        c = lax.axis_index("c")
        s = lax.axis_index("s")
        wid = c * NS + s
        pltpu.sync_copy(zeros_hbm, vbuf)
        pltpu.sync_copy(vbuf, acc.at[pl.ds(s * ZR, ZR)])
        pltpu.sync_copy(ones_hbm, ones_v)
        plsc.subcore_barrier()
        base = wid * (E_PAD // NW)

        def body(j, carry):
            off = base + j * CHK
            pltpu.sync_copy(dst_hbm.at[pl.ds(off, CHK)], didx)
            pltpu.sync_copy(ones_v, acc.at[didx], add=True)
            return carry

        lax.fori_loop(0, NCH, body, 0)
        plsc.subcore_barrier()
        pltpu.sync_copy(acc.at[pl.ds(s * ZR, ZR)], vbuf)
        pltpu.sync_copy(vbuf, out_hbm.at[pl.ds(c * NP + s * ZR, ZR)])

    return deg_kernel


INNER = 8


@functools.lru_cache(maxsize=None)
def _make_prop():
    @functools.partial(
        pl.kernel,
        out_type=jax.ShapeDtypeStruct((NP, C), jnp.float32),
        mesh=_mesh(),
        scratch_types=[
            pltpu.VMEM_SHARED((NP, C), jnp.float32),
            pltpu.VMEM((NCHA // 2, CHK), jnp.int32),
            pltpu.VMEM((NCHA // 2, CHK), jnp.int32),
            pltpu.VMEM((CHK, C), jnp.float32),
            pltpu.VMEM((CHK, C), jnp.float32),
            pltpu.SemaphoreType.DMA,
            pltpu.SemaphoreType.DMA,
        ],
    )
    def prop_kernel(srcm_hbm, dstm_hbm, y_hbm, zeros_hbm, out_hbm,
                    acc, sidx, didx, rows0, rows1, sg0, sg1):
        c = lax.axis_index("c")
        s = lax.axis_index("s")

        rows = (rows0, rows1)
        sems = (sg0, sg1)

        def seg(kk, carry):
            c0 = kk * INNER
            cp = [pltpu.async_copy(y_hbm.at[sidx.at[c0]], rows0, sg0),
                  pltpu.async_copy(y_hbm.at[sidx.at[c0 + 1]], rows1, sg1)]
            for jj in range(INNER):
                b = jj % 2
                cp[b].wait()
                pltpu.sync_copy(rows[b], acc.at[didx.at[c0 + jj]], add=True)
                if jj + 2 < INNER:
                    cp[b] = pltpu.async_copy(y_hbm.at[sidx.at[c0 + jj + 2]],
                                             rows[b], sems[b])
            return carry

        @pl.when(c == 0)
        def _():
            pltpu.sync_copy(zeros_hbm, rows0)
            for k in range(ZR // CHK):
                pltpu.sync_copy(rows0, acc.at[pl.ds(s * ZR + k * CHK, CHK)])
            plsc.subcore_barrier()
            halfc = NCHA // 2
            for ph in range(2):
                pltpu.sync_copy(
                    srcm_hbm.at[pl.ds(s * NCHA + ph * halfc, halfc)], sidx)
                pltpu.sync_copy(
                    dstm_hbm.at[pl.ds(s * NCHA + ph * halfc, halfc)], didx)
                lax.fori_loop(0, halfc // INNER, seg, 0)
            plsc.subcore_barrier()
            wb = [None, None]
            for k in range(ZR // CHK):
                b = k % 2
                if wb[b] is not None:
                    wb[b].wait()
                pltpu.sync_copy(acc.at[pl.ds(s * ZR + k * CHK, CHK)], rows[b])
                wb[b] = pltpu.async_copy(
                    rows[b], out_hbm.at[pl.ds(s * ZR + k * CHK, CHK)], sems[b])
            wb[0].wait()
            wb[1].wait()

    return prop_kernel


BM = 2000


def _k1_body(x_ref, w_ref, degp_ref, z1_ref, dis_ref):
    deg = degp_ref[0] + degp_ref[1] + 1.0
    dis = lax.rsqrt(deg)
    z1_ref[...] = jnp.dot(x_ref[...], w_ref[...],
                          preferred_element_type=jnp.float32) * dis
    dis_ref[...] = dis


def _k1(x, w, degp):
    return pl.pallas_call(
        _k1_body,
        grid=(N // BM,),
        in_specs=[
            pl.BlockSpec((BM, F), lambda i: (i, 0)),
            pl.BlockSpec((F, C), lambda i: (0, 0)),
            pl.BlockSpec((NC, BM, 1), lambda i: (0, i, 0)),
        ],
        out_specs=[
            pl.BlockSpec((BM, C), lambda i: (i, 0)),
            pl.BlockSpec((BM, 1), lambda i: (i, 0)),
        ],
        out_shape=[
            jax.ShapeDtypeStruct((N, C), jnp.float32),
            jax.ShapeDtypeStruct((N, 1), jnp.float32),
        ],
    )(x, w, degp)


def _k2_body(p_ref, z_ref, dis_ref, b_ref, w_ref, out_ref):
    dis = dis_ref[...]
    m = dis * (p_ref[...] + z_ref[...])
    h = jnp.maximum(m[:, :H] + b_ref[...], 0.0)
    out_ref[...] = jnp.dot(h, w_ref[...], preferred_element_type=jnp.float32) * dis


def _k2(p, z, dis, b, w):
    return pl.pallas_call(
        _k2_body,
        grid=(N // BM,),
        in_specs=[
            pl.BlockSpec((BM, C), lambda i: (i, 0)),
            pl.BlockSpec((BM, C), lambda i: (i, 0)),
            pl.BlockSpec((BM, 1), lambda i: (i, 0)),
            pl.BlockSpec((1, H), lambda i: (0, 0)),
            pl.BlockSpec((H, C), lambda i: (0, 0)),
        ],
        out_specs=pl.BlockSpec((BM, C), lambda i: (i, 0)),
        out_shape=jax.ShapeDtypeStruct((N, C), jnp.float32),
    )(p, z, dis, b, w)


def _k3_body(r_ref, z_ref, dis_ref, b3_ref, b5_ref, z4_ref, s_ref):
    dis = dis_ref[...]
    m = dis * (r_ref[...] + z_ref[...])
    a = jnp.maximum(m[:, :H] + b3_ref[...], 0.0)
    z4_ref[...] = jnp.concatenate([dis * a, jnp.zeros_like(a)], axis=1)
    s_ref[...] = jnp.maximum(m[:, H:] + b5_ref[...], 0.0)


def _k3(r, z35, dis, b3, b5):
    return pl.pallas_call(
        _k3_body,
        grid=(N // BM,),
        in_specs=[
            pl.BlockSpec((BM, C), lambda i: (i, 0)),
            pl.BlockSpec((BM, C), lambda i: (i, 0)),
            pl.BlockSpec((BM, 1), lambda i: (i, 0)),
            pl.BlockSpec((1, H), lambda i: (0, 0)),
            pl.BlockSpec((1, H), lambda i: (0, 0)),
        ],
        out_specs=[
            pl.BlockSpec((BM, C), lambda i: (i, 0)),
            pl.BlockSpec((BM, H), lambda i: (i, 0)),
        ],
        out_shape=[
            jax.ShapeDtypeStruct((N, C), jnp.float32),
            jax.ShapeDtypeStruct((N, H), jnp.float32),
        ],
    )(r, z35, dis, b3, b5)


def _k4_body(t_ref, z_ref, dis_ref, w_ref, b_ref, out_ref):
    dis = dis_ref[...]
    xin = (dis * (t_ref[...] + z_ref[...]))[:, :H]
    out_ref[...] = jnp.maximum(
        jnp.dot(xin, w_ref[...], preferred_element_type=jnp.float32) + b_ref[...],
        0.0)


def _k4(t, z4, dis, w, b):
    return pl.pallas_call(
        _k4_body,
        grid=(N // BM,),
        in_specs=[
            pl.BlockSpec((BM, C), lambda i: (i, 0)),
            pl.BlockSpec((BM, C), lambda i: (i, 0)),
            pl.BlockSpec((BM, 1), lambda i: (i, 0)),
            pl.BlockSpec((H, F), lambda i: (0, 0)),
            pl.BlockSpec((1, F), lambda i: (0, 0)),
        ],
        out_specs=pl.BlockSpec((BM, F), lambda i: (i, 0)),
        out_shape=jax.ShapeDtypeStruct((N, F), jnp.float32),
    )(t, z4, dis, w, b)


BA = 400


def _k5_body(si_ref, sj_ref, out_ref):
    out_ref[...] = lax.dot_general(
        si_ref[...], sj_ref[...],
        dimension_numbers=(((1,), (1,)), ((), ())),
        preferred_element_type=jnp.float32)


def _k5(s):
    return pl.pallas_call(
        _k5_body,
        grid=(N // BA,),
        in_specs=[
            pl.BlockSpec((BA, H), lambda i: (i, 0)),
            pl.BlockSpec((N, H), lambda i: (0, 0)),
        ],
        out_specs=pl.BlockSpec((BA, N), lambda i: (i, 0)),
        out_shape=jax.ShapeDtypeStruct((N, N), jnp.float32),
    )(s, s)


def kernel(x, edge_index, W_enc1, b_enc1, W_enc2, b_enc2,
           W_attr1, b_attr1, W_attr2, b_attr2, W_struct, b_struct):
    src = edge_index[0].astype(jnp.int32)
    dst = edge_index[1].astype(jnp.int32)
    pad = E_PAD - E
    src_p = jnp.concatenate([src, jnp.zeros((pad,), jnp.int32)])
    dst_p = jnp.concatenate([dst, jnp.full((pad,), N, jnp.int32)])

    zeros1 = jnp.zeros((ZR,), jnp.float32)
    zerosC = jnp.zeros((CHK, C), jnp.float32)
    ones = jnp.ones((CHK,), jnp.float32)

    wpad = jnp.zeros((H, C - H), jnp.float32)
    w1p = jnp.concatenate([W_enc1, jnp.zeros((F, C - H), jnp.float32)], axis=1)
    w2p = jnp.concatenate([W_enc2, wpad], axis=1)
    w35 = jnp.concatenate([W_attr1, W_struct], axis=1)

    srcm = src_p.reshape(E_PAD // CHK, CHK)
    dstm = dst_p.reshape(E_PAD // CHK, CHK)

    prop = _make_prop()
    degp = _make_deg()(dst_p, ones, zeros1).reshape(NC, NP, 1)
    z1, dis = _k1(x, w1p, degp)

    p = prop(srcm, dstm, z1, zerosC)
    z2 = _k2(p, z1, dis, b_enc1.reshape(1, -1), w2p)

    q = prop(srcm, dstm, z2, zerosC)
    z35 = _k2(q, z2, dis, b_enc2.reshape(1, -1), w35)

    r = prop(srcm, dstm, z35, zerosC)
    z4, s = _k3(r, z35, dis, b_attr1.reshape(1, -1), b_struct.reshape(1, -1))

    t = prop(srcm, dstm, z4, zerosC)
    X_hat = _k4(t, z4, dis, W_attr2, b_attr2.reshape(1, -1))

    A_hat = _k5(s)
    return (A_hat, X_hat)


# speedup vs baseline: 1.2372x; 1.2372x over previous
import functools

import jax
import jax.numpy as jnp
from jax import lax
from jax.experimental import pallas as pl
from jax.experimental.pallas import tpu as pltpu
from jax.experimental.pallas import tpu_sc as plsc

N = 10000
F = 128
H = 64
E = 160000
C = 128

NC = 2
NS = 16
NW = NC * NS
CHK = 128
NCH = 40
NCH0 = 64
NCH1 = 16
E_PAD = NW * CHK * NCH
NP = 10240
ZR = NP // NS

_mesh = lambda: plsc.VectorSubcoreMesh(core_axis_name="c", subcore_axis_name="s",
                                       num_cores=NC, num_subcores=NS)


@functools.lru_cache(maxsize=None)
def _make_deg():
    @functools.partial(
        pl.kernel,
        out_type=jax.ShapeDtypeStruct((NC * NP,), jnp.float32),
        mesh=_mesh(),
        scratch_types=[
            pltpu.VMEM_SHARED((NP,), jnp.float32),
            pltpu.VMEM((CHK,), jnp.int32),
            pltpu.VMEM((CHK,), jnp.float32),
            pltpu.VMEM((ZR,), jnp.float32),
        ],
    )
    def deg_kernel(dst_hbm, ones_hbm, zeros_hbm, out_hbm, acc, didx, ones_v, vbuf):
        c = lax.axis_index("c")
        s = lax.axis_index("s")
        wid = c * NS + s
        pltpu.sync_copy(zeros_hbm, vbuf)
        pltpu.sync_copy(vbuf, acc.at[pl.ds(s * ZR, ZR)])
        pltpu.sync_copy(ones_hbm, ones_v)
        plsc.subcore_barrier()
        base = wid * (E_PAD // NW)

        def body(j, carry):
            off = base + j * CHK
            pltpu.sync_copy(dst_hbm.at[pl.ds(off, CHK)], didx)
            pltpu.sync_copy(ones_v, acc.at[didx], add=True)
            return carry

        lax.fori_loop(0, NCH, body, 0)
        plsc.subcore_barrier()
        pltpu.sync_copy(acc.at[pl.ds(s * ZR, ZR)], vbuf)
        pltpu.sync_copy(vbuf, out_hbm.at[pl.ds(c * NP + s * ZR, ZR)])

    return deg_kernel


INNER = 8


@functools.lru_cache(maxsize=None)
def _make_prop():
    @functools.partial(
        pl.kernel,
        out_type=jax.ShapeDtypeStruct((NC * NP, C), jnp.float32),
        mesh=_mesh(),
        scratch_types=[
            pltpu.VMEM_SHARED((NP, C), jnp.float32),
            pltpu.VMEM((NCH0 // 2, CHK), jnp.int32),
            pltpu.VMEM((NCH0 // 2, CHK), jnp.int32),
            pltpu.VMEM((CHK, C), jnp.float32),
            pltpu.VMEM((CHK, C), jnp.float32),
            pltpu.SemaphoreType.DMA,
            pltpu.SemaphoreType.DMA,
        ],
    )
    def prop_kernel(srcm_hbm, dstm_hbm, y_hbm, zeros_hbm, out_hbm,
                    acc, sidx, didx, rows0, rows1, sg0, sg1):
        c = lax.axis_index("c")
        s = lax.axis_index("s")
        pltpu.sync_copy(zeros_hbm, rows0)
        for k in range(ZR // CHK):
            pltpu.sync_copy(rows0, acc.at[pl.ds(s * ZR + k * CHK, CHK)])
        plsc.subcore_barrier()

        rows = (rows0, rows1)
        sems = (sg0, sg1)

        def seg(kk, carry):
            c0 = kk * INNER
            cp = [pltpu.async_copy(y_hbm.at[sidx.at[c0]], rows0, sg0),
                  pltpu.async_copy(y_hbm.at[sidx.at[c0 + 1]], rows1, sg1)]
            for jj in range(INNER):
                b = jj % 2
                cp[b].wait()
                pltpu.sync_copy(rows[b], acc.at[didx.at[c0 + jj]], add=True)
                if jj + 2 < INNER:
                    cp[b] = pltpu.async_copy(y_hbm.at[sidx.at[c0 + jj + 2]],
                                             rows[b], sems[b])
            return carry

        def run_edges(base_chunk, nch_c):
            halfc = nch_c // 2
            for ph in range(2):
                pltpu.sync_copy(
                    srcm_hbm.at[pl.ds(base_chunk + ph * halfc, halfc)],
                    sidx.at[pl.ds(0, halfc)])
                pltpu.sync_copy(
                    dstm_hbm.at[pl.ds(base_chunk + ph * halfc, halfc)],
                    didx.at[pl.ds(0, halfc)])
                lax.fori_loop(0, halfc // INNER, seg, 0)

        @pl.when(c == 0)
        def _():
            run_edges(s * NCH0, NCH0)

        @pl.when(c == 1)
        def _():
            run_edges(NS * NCH0 + s * NCH1, NCH1)

        plsc.subcore_barrier()

        wb = [None, None]
        for k in range(ZR // CHK):
            b = k % 2
            if wb[b] is not None:
                wb[b].wait()
            pltpu.sync_copy(acc.at[pl.ds(s * ZR + k * CHK, CHK)], rows[b])
            wb[b] = pltpu.async_copy(
                rows[b], out_hbm.at[pl.ds(c * NP + s * ZR + k * CHK, CHK)], sems[b])
        wb[0].wait()
        wb[1].wait()

    return prop_kernel


BM = 2000


def _k1_body(x_ref, w_ref, degp_ref, z1_ref, dis_ref):
    deg = degp_ref[0] + degp_ref[1] + 1.0
    dis = lax.rsqrt(deg)
    z1_ref[...] = jnp.dot(x_ref[...], w_ref[...],
                          preferred_element_type=jnp.float32) * dis
    dis_ref[...] = dis


def _k1(x, w, degp):
    return pl.pallas_call(
        _k1_body,
        grid=(N // BM,),
        in_specs=[
            pl.BlockSpec((BM, F), lambda i: (i, 0)),
            pl.BlockSpec((F, C), lambda i: (0, 0)),
            pl.BlockSpec((NC, BM, 1), lambda i: (0, i, 0)),
        ],
        out_specs=[
            pl.BlockSpec((BM, C), lambda i: (i, 0)),
            pl.BlockSpec((BM, 1), lambda i: (i, 0)),
        ],
        out_shape=[
            jax.ShapeDtypeStruct((N, C), jnp.float32),
            jax.ShapeDtypeStruct((N, 1), jnp.float32),
        ],
    )(x, w, degp)


def _k2_body(p_ref, z_ref, dis_ref, b_ref, w_ref, out_ref):
    dis = dis_ref[...]
    m = dis * (p_ref[0] + p_ref[1] + z_ref[...])
    h = jnp.maximum(m[:, :H] + b_ref[...], 0.0)
    out_ref[...] = jnp.dot(h, w_ref[...], preferred_element_type=jnp.float32) * dis


def _k2(p, z, dis, b, w):
    return pl.pallas_call(
        _k2_body,
        grid=(N // BM,),
        in_specs=[
            pl.BlockSpec((NC, BM, C), lambda i: (0, i, 0)),
            pl.BlockSpec((BM, C), lambda i: (i, 0)),
            pl.BlockSpec((BM, 1), lambda i: (i, 0)),
            pl.BlockSpec((1, H), lambda i: (0, 0)),
            pl.BlockSpec((H, C), lambda i: (0, 0)),
        ],
        out_specs=pl.BlockSpec((BM, C), lambda i: (i, 0)),
        out_shape=jax.ShapeDtypeStruct((N, C), jnp.float32),
    )(p, z, dis, b, w)


def _k3_body(r_ref, z_ref, dis_ref, b3_ref, b5_ref, z4_ref, s_ref):
    dis = dis_ref[...]
    m = dis * (r_ref[0] + r_ref[1] + z_ref[...])
    a = jnp.maximum(m[:, :H] + b3_ref[...], 0.0)
    z4_ref[...] = jnp.concatenate([dis * a, jnp.zeros_like(a)], axis=1)
    s_ref[...] = jnp.maximum(m[:, H:] + b5_ref[...], 0.0)


def _k3(r, z35, dis, b3, b5):
    return pl.pallas_call(
        _k3_body,
        grid=(N // BM,),
        in_specs=[
            pl.BlockSpec((NC, BM, C), lambda i: (0, i, 0)),
            pl.BlockSpec((BM, C), lambda i: (i, 0)),
            pl.BlockSpec((BM, 1), lambda i: (i, 0)),
            pl.BlockSpec((1, H), lambda i: (0, 0)),
            pl.BlockSpec((1, H), lambda i: (0, 0)),
        ],
        out_specs=[
            pl.BlockSpec((BM, C), lambda i: (i, 0)),
            pl.BlockSpec((BM, H), lambda i: (i, 0)),
        ],
        out_shape=[
            jax.ShapeDtypeStruct((N, C), jnp.float32),
            jax.ShapeDtypeStruct((N, H), jnp.float32),
        ],
    )(r, z35, dis, b3, b5)


def _k4_body(t_ref, z_ref, dis_ref, w_ref, b_ref, out_ref):
    dis = dis_ref[...]
    xin = (dis * (t_ref[0] + t_ref[1] + z_ref[...]))[:, :H]
    out_ref[...] = jnp.maximum(
        jnp.dot(xin, w_ref[...], preferred_element_type=jnp.float32) + b_ref[...],
        0.0)


def _k4(t, z4, dis, w, b):
    return pl.pallas_call(
        _k4_body,
        grid=(N // BM,),
        in_specs=[
            pl.BlockSpec((NC, BM, C), lambda i: (0, i, 0)),
            pl.BlockSpec((BM, C), lambda i: (i, 0)),
            pl.BlockSpec((BM, 1), lambda i: (i, 0)),
            pl.BlockSpec((H, F), lambda i: (0, 0)),
            pl.BlockSpec((1, F), lambda i: (0, 0)),
        ],
        out_specs=pl.BlockSpec((BM, F), lambda i: (i, 0)),
        out_shape=jax.ShapeDtypeStruct((N, F), jnp.float32),
    )(t, z4, dis, w, b)


BA = 400


def _k5_body(si_ref, sj_ref, out_ref):
    out_ref[...] = lax.dot_general(
        si_ref[...], sj_ref[...],
        dimension_numbers=(((1,), (1,)), ((), ())),
        preferred_element_type=jnp.float32)


def _k5(s):
    return pl.pallas_call(
        _k5_body,
        grid=(N // BA,),
        in_specs=[
            pl.BlockSpec((BA, H), lambda i: (i, 0)),
            pl.BlockSpec((N, H), lambda i: (0, 0)),
        ],
        out_specs=pl.BlockSpec((BA, N), lambda i: (i, 0)),
        out_shape=jax.ShapeDtypeStruct((N, N), jnp.float32),
    )(s, s)


def kernel(x, edge_index, W_enc1, b_enc1, W_enc2, b_enc2,
           W_attr1, b_attr1, W_attr2, b_attr2, W_struct, b_struct):
    src = edge_index[0].astype(jnp.int32)
    dst = edge_index[1].astype(jnp.int32)
    pad = E_PAD - E
    src_p = jnp.concatenate([src, jnp.zeros((pad,), jnp.int32)])
    dst_p = jnp.concatenate([dst, jnp.full((pad,), N, jnp.int32)])

    zeros1 = jnp.zeros((ZR,), jnp.float32)
    zerosC = jnp.zeros((CHK, C), jnp.float32)
    ones = jnp.ones((CHK,), jnp.float32)

    wpad = jnp.zeros((H, C - H), jnp.float32)
    w1p = jnp.concatenate([W_enc1, jnp.zeros((F, C - H), jnp.float32)], axis=1)
    w2p = jnp.concatenate([W_enc2, wpad], axis=1)
    w35 = jnp.concatenate([W_attr1, W_struct], axis=1)

    srcm = src_p.reshape(E_PAD // CHK, CHK)
    dstm = dst_p.reshape(E_PAD // CHK, CHK)

    prop = _make_prop()
    degp = _make_deg()(dst_p, ones, zeros1).reshape(NC, NP, 1)
    z1, dis = _k1(x, w1p, degp)

    p = prop(srcm, dstm, z1, zerosC).reshape(NC, NP, C)
    z2 = _k2(p, z1, dis, b_enc1.reshape(1, -1), w2p)

    q = prop(srcm, dstm, z2, zerosC).reshape(NC, NP, C)
    z35 = _k2(q, z2, dis, b_enc2.reshape(1, -1), w35)

    r = prop(srcm, dstm, z35, zerosC).reshape(NC, NP, C)
    z4, s = _k3(r, z35, dis, b_attr1.reshape(1, -1), b_struct.reshape(1, -1))

    t = prop(srcm, dstm, z4, zerosC).reshape(NC, NP, C)
    X_hat = _k4(t, z4, dis, W_attr2, b_attr2.reshape(1, -1))

    A_hat = _k5(s)
    return (A_hat, X_hat)


# speedup vs baseline: 1.7549x; 1.4185x over previous
import functools

import jax
import jax.numpy as jnp
from jax import lax
from jax.experimental import pallas as pl
from jax.experimental.pallas import tpu as pltpu
from jax.experimental.pallas import tpu_sc as plsc

N = 10000
F = 128
H = 64
E = 160000
C = 128

NC = 2
NS = 16
NW = NC * NS
CHK = 128
NCH = 40
NCH0 = 64
NCH1 = 16
E_PAD = NW * CHK * NCH
NP = 10240
ZR = NP // NS

_mesh = lambda: plsc.VectorSubcoreMesh(core_axis_name="c", subcore_axis_name="s",
                                       num_cores=NC, num_subcores=NS)


@functools.lru_cache(maxsize=None)
def _make_deg():
    @functools.partial(
        pl.kernel,
        out_type=jax.ShapeDtypeStruct((NC * NP,), jnp.float32),
        mesh=_mesh(),
        scratch_types=[
            pltpu.VMEM_SHARED((NP,), jnp.float32),
            pltpu.VMEM((CHK,), jnp.int32),
            pltpu.VMEM((CHK,), jnp.float32),
            pltpu.VMEM((ZR,), jnp.float32),
        ],
    )
    def deg_kernel(dst_hbm, ones_hbm, zeros_hbm, out_hbm, acc, didx, ones_v, vbuf):
        c = lax.axis_index("c")
        s = lax.axis_index("s")
        wid = c * NS + s
        pltpu.sync_copy(zeros_hbm, vbuf)
        pltpu.sync_copy(vbuf, acc.at[pl.ds(s * ZR, ZR)])
        pltpu.sync_copy(ones_hbm, ones_v)
        plsc.subcore_barrier()
        base = wid * (E_PAD // NW)

        def body(j, carry):
            off = base + j * CHK
            pltpu.sync_copy(dst_hbm.at[pl.ds(off, CHK)], didx)
            pltpu.sync_copy(ones_v, acc.at[didx], add=True)
            return carry

        lax.fori_loop(0, NCH, body, 0)
        plsc.subcore_barrier()
        pltpu.sync_copy(acc.at[pl.ds(s * ZR, ZR)], vbuf)
        pltpu.sync_copy(vbuf, out_hbm.at[pl.ds(c * NP + s * ZR, ZR)])

    return deg_kernel


INNER = 8


@functools.lru_cache(maxsize=None)
def _make_prop(W):
    @functools.partial(
        pl.kernel,
        out_type=jax.ShapeDtypeStruct((NC * NP, W), jnp.float32),
        mesh=_mesh(),
        scratch_types=[
            pltpu.VMEM_SHARED((NP, W), jnp.float32),
            pltpu.VMEM((NCH0 // 2, CHK), jnp.int32),
            pltpu.VMEM((NCH0 // 2, CHK), jnp.int32),
            pltpu.VMEM((CHK, W), jnp.float32),
            pltpu.VMEM((CHK, W), jnp.float32),
            pltpu.SemaphoreType.DMA,
            pltpu.SemaphoreType.DMA,
        ],
        compiler_params=pltpu.CompilerParams(use_tc_tiling_on_sc=False),
    )
    def prop_kernel(srcm_hbm, dstm_hbm, y_hbm, zeros_hbm, out_hbm,
                    acc, sidx, didx, rows0, rows1, sg0, sg1):
        c = lax.axis_index("c")
        s = lax.axis_index("s")
        pltpu.sync_copy(zeros_hbm, rows0)
        for k in range(ZR // CHK):
            pltpu.sync_copy(rows0, acc.at[pl.ds(s * ZR + k * CHK, CHK)])
        plsc.subcore_barrier()

        rows = (rows0, rows1)
        sems = (sg0, sg1)

        def seg(kk, carry):
            c0 = kk * INNER
            cp = [pltpu.async_copy(y_hbm.at[sidx.at[c0]], rows0, sg0),
                  pltpu.async_copy(y_hbm.at[sidx.at[c0 + 1]], rows1, sg1)]
            for jj in range(INNER):
                b = jj % 2
                cp[b].wait()
                pltpu.sync_copy(rows[b], acc.at[didx.at[c0 + jj]], add=True)
                if jj + 2 < INNER:
                    cp[b] = pltpu.async_copy(y_hbm.at[sidx.at[c0 + jj + 2]],
                                             rows[b], sems[b])
            return carry

        def run_edges(base_chunk, nch_c):
            halfc = nch_c // 2
            for ph in range(2):
                pltpu.sync_copy(
                    srcm_hbm.at[pl.ds(base_chunk + ph * halfc, halfc)],
                    sidx.at[pl.ds(0, halfc)])
                pltpu.sync_copy(
                    dstm_hbm.at[pl.ds(base_chunk + ph * halfc, halfc)],
                    didx.at[pl.ds(0, halfc)])
                lax.fori_loop(0, halfc // INNER, seg, 0)

        @pl.when(c == 0)
        def _():
            run_edges(s * NCH0, NCH0)

        @pl.when(c == 1)
        def _():
            run_edges(NS * NCH0 + s * NCH1, NCH1)

        plsc.subcore_barrier()

        wb = [None, None]
        for k in range(ZR // CHK):
            b = k % 2
            if wb[b] is not None:
                wb[b].wait()
            pltpu.sync_copy(acc.at[pl.ds(s * ZR + k * CHK, CHK)], rows[b])
            wb[b] = pltpu.async_copy(
                rows[b], out_hbm.at[pl.ds(c * NP + s * ZR + k * CHK, CHK)], sems[b])
        wb[0].wait()
        wb[1].wait()

    return prop_kernel


BM = 2000


def _k1_body(x_ref, w_ref, degp_ref, z1_ref, dis_ref):
    deg = degp_ref[0] + degp_ref[1] + 1.0
    dis = lax.rsqrt(deg)
    z1_ref[...] = jnp.dot(x_ref[...], w_ref[...],
                          preferred_element_type=jnp.float32) * dis
    dis_ref[...] = dis


def _k1(x, w, degp):
    return pl.pallas_call(
        _k1_body,
        grid=(N // BM,),
        in_specs=[
            pl.BlockSpec((BM, F), lambda i: (i, 0)),
            pl.BlockSpec((F, H), lambda i: (0, 0)),
            pl.BlockSpec((NC, BM, 1), lambda i: (0, i, 0)),
        ],
        out_specs=[
            pl.BlockSpec((BM, H), lambda i: (i, 0)),
            pl.BlockSpec((BM, 1), lambda i: (i, 0)),
        ],
        out_shape=[
            jax.ShapeDtypeStruct((N, H), jnp.float32),
            jax.ShapeDtypeStruct((N, 1), jnp.float32),
        ],
    )(x, w, degp)


def _k2_body(p_ref, z_ref, dis_ref, b_ref, w_ref, out_ref):
    dis = dis_ref[...]
    m = dis * (p_ref[0] + p_ref[1] + z_ref[...])
    h = jnp.maximum(m + b_ref[...], 0.0)
    out_ref[...] = jnp.dot(h, w_ref[...], preferred_element_type=jnp.float32) * dis


def _k2(p, z, dis, b, w):
    c_out = w.shape[1]
    return pl.pallas_call(
        _k2_body,
        grid=(N // BM,),
        in_specs=[
            pl.BlockSpec((NC, BM, H), lambda i: (0, i, 0)),
            pl.BlockSpec((BM, H), lambda i: (i, 0)),
            pl.BlockSpec((BM, 1), lambda i: (i, 0)),
            pl.BlockSpec((1, H), lambda i: (0, 0)),
            pl.BlockSpec((H, c_out), lambda i: (0, 0)),
        ],
        out_specs=pl.BlockSpec((BM, c_out), lambda i: (i, 0)),
        out_shape=jax.ShapeDtypeStruct((N, c_out), jnp.float32),
    )(p, z, dis, b, w)


def _k3_body(r_ref, z_ref, dis_ref, b3_ref, b5_ref, z4_ref, s_ref):
    dis = dis_ref[...]
    m = dis * (r_ref[0] + r_ref[1] + z_ref[...])
    a = jnp.maximum(m[:, :H] + b3_ref[...], 0.0)
    z4_ref[...] = dis * a
    s_ref[...] = jnp.maximum(m[:, H:] + b5_ref[...], 0.0)


def _k3(r, z35, dis, b3, b5):
    return pl.pallas_call(
        _k3_body,
        grid=(N // BM,),
        in_specs=[
            pl.BlockSpec((NC, BM, 2 * H), lambda i: (0, i, 0)),
            pl.BlockSpec((BM, 2 * H), lambda i: (i, 0)),
            pl.BlockSpec((BM, 1), lambda i: (i, 0)),
            pl.BlockSpec((1, H), lambda i: (0, 0)),
            pl.BlockSpec((1, H), lambda i: (0, 0)),
        ],
        out_specs=[
            pl.BlockSpec((BM, H), lambda i: (i, 0)),
            pl.BlockSpec((BM, H), lambda i: (i, 0)),
        ],
        out_shape=[
            jax.ShapeDtypeStruct((N, H), jnp.float32),
            jax.ShapeDtypeStruct((N, H), jnp.float32),
        ],
    )(r, z35, dis, b3, b5)


def _k4_body(t_ref, z_ref, dis_ref, w_ref, b_ref, out_ref):
    dis = dis_ref[...]
    xin = dis * (t_ref[0] + t_ref[1] + z_ref[...])
    out_ref[...] = jnp.maximum(
        jnp.dot(xin, w_ref[...], preferred_element_type=jnp.float32) + b_ref[...],
        0.0)


def _k4(t, z4, dis, w, b):
    return pl.pallas_call(
        _k4_body,
        grid=(N // BM,),
        in_specs=[
            pl.BlockSpec((NC, BM, H), lambda i: (0, i, 0)),
            pl.BlockSpec((BM, H), lambda i: (i, 0)),
            pl.BlockSpec((BM, 1), lambda i: (i, 0)),
            pl.BlockSpec((H, F), lambda i: (0, 0)),
            pl.BlockSpec((1, F), lambda i: (0, 0)),
        ],
        out_specs=pl.BlockSpec((BM, F), lambda i: (i, 0)),
        out_shape=jax.ShapeDtypeStruct((N, F), jnp.float32),
    )(t, z4, dis, w, b)


BA = 400


def _k5_body(si_ref, sj_ref, out_ref):
    out_ref[...] = lax.dot_general(
        si_ref[...], sj_ref[...],
        dimension_numbers=(((1,), (1,)), ((), ())),
        preferred_element_type=jnp.float32)


def _k5(s):
    return pl.pallas_call(
        _k5_body,
        grid=(N // BA,),
        in_specs=[
            pl.BlockSpec((BA, H), lambda i: (i, 0)),
            pl.BlockSpec((N, H), lambda i: (0, 0)),
        ],
        out_specs=pl.BlockSpec((BA, N), lambda i: (i, 0)),
        out_shape=jax.ShapeDtypeStruct((N, N), jnp.float32),
    )(s, s)


def kernel(x, edge_index, W_enc1, b_enc1, W_enc2, b_enc2,
           W_attr1, b_attr1, W_attr2, b_attr2, W_struct, b_struct):
    src = edge_index[0].astype(jnp.int32)
    dst = edge_index[1].astype(jnp.int32)
    pad = E_PAD - E
    src_p = jnp.concatenate([src, jnp.zeros((pad,), jnp.int32)])
    dst_p = jnp.concatenate([dst, jnp.full((pad,), N, jnp.int32)])

    zeros1 = jnp.zeros((ZR,), jnp.float32)
    zeros64 = jnp.zeros((CHK, H), jnp.float32)
    zeros128 = jnp.zeros((CHK, 2 * H), jnp.float32)
    ones = jnp.ones((CHK,), jnp.float32)

    w35 = jnp.concatenate([W_attr1, W_struct], axis=1)

    srcm = src_p.reshape(E_PAD // CHK, CHK)
    dstm = dst_p.reshape(E_PAD // CHK, CHK)

    prop64 = _make_prop(H)
    prop128 = _make_prop(2 * H)
    degp = _make_deg()(dst_p, ones, zeros1).reshape(NC, NP, 1)
    z1, dis = _k1(x, W_enc1, degp)

    p = prop64(srcm, dstm, z1, zeros64).reshape(NC, NP, H)
    z2 = _k2(p, z1, dis, b_enc1.reshape(1, -1), W_enc2)

    q = prop64(srcm, dstm, z2, zeros64).reshape(NC, NP, H)
    z35 = _k2(q, z2, dis, b_enc2.reshape(1, -1), w35)

    r = prop128(srcm, dstm, z35, zeros128).reshape(NC, NP, 2 * H)
    z4, s = _k3(r, z35, dis, b_attr1.reshape(1, -1), b_struct.reshape(1, -1))

    t = prop64(srcm, dstm, z4, zeros64).reshape(NC, NP, H)
    X_hat = _k4(t, z4, dis, W_attr2, b_attr2.reshape(1, -1))

    A_hat = _k5(s)
    return (A_hat, X_hat)
